# baseline (speedup 1.0000x reference)
import functools

import jax
import jax.numpy as jnp
from jax import lax
from jax.experimental import pallas as pl
from jax.experimental.pallas import tpu as pltpu
from jax.experimental.pallas import tpu_sc as plsc

EDGE_VOCAB = 64
EDGE_DIM = 128
HIDDEN_DIM = 256
B, N = 16, 64
B_TOT = B * N * N


def _mlp_table_kernel(table_ref, w1_ref, b1_ref, w2_ref, b2_ref, out_ref):
    h = jnp.dot(table_ref[...], w1_ref[...], preferred_element_type=jnp.float32)
    h = h + b1_ref[...]
    h = jax.nn.gelu(h)
    o = jnp.dot(h, w2_ref[...], preferred_element_type=jnp.float32)
    out_ref[...] = o + b2_ref[...]


def _fused_table(table, W1, b1, W2, b2):
    return pl.pallas_call(
        _mlp_table_kernel,
        out_shape=jax.ShapeDtypeStruct((EDGE_VOCAB, HIDDEN_DIM), jnp.float32),
    )(table, W1, b1.reshape(1, HIDDEN_DIM), W2, b2.reshape(1, HIDDEN_DIM))


def _make_gather():
    info = plsc.get_sparse_core_info()
    NC, NS = info.num_cores, info.num_subcores
    NW = NC * NS
    b_per_w = B_TOT // NW
    CHUNK = 128
    n_chunks = b_per_w // CHUNK
    mesh = plsc.VectorSubcoreMesh(core_axis_name="c", subcore_axis_name="s")

    @functools.partial(
        pl.kernel,
        mesh=mesh,
        out_type=jax.ShapeDtypeStruct((B_TOT, HIDDEN_DIM), jnp.float32),
        scratch_types=[
            pltpu.VMEM((b_per_w,), jnp.int32),
            pltpu.VMEM((2, CHUNK, HIDDEN_DIM), jnp.float32),
            pltpu.SemaphoreType.DMA,
            pltpu.SemaphoreType.DMA,
        ],
    )
    def gather_k(idx_hbm, table_hbm, out_hbm, idx_v, rows_v, sem0, sem1):
        wid = lax.axis_index("s") * NC + lax.axis_index("c")
        base = wid * b_per_w
        sems = (sem0, sem1)
        pltpu.sync_copy(idx_hbm.at[pl.ds(base, b_per_w)], idx_v)

        def start_gather(i):
            return pltpu.async_copy(
                table_hbm.at[idx_v.at[pl.ds(i * CHUNK, CHUNK)]],
                rows_v.at[i % 2],
                sems[i % 2],
            )

        handles = {0: start_gather(0)}
        for i in range(n_chunks):
            handles.pop(i).wait()
            if i + 1 < n_chunks:
                handles[i + 1] = start_gather(i + 1)
            pltpu.sync_copy(
                rows_v.at[i % 2], out_hbm.at[pl.ds(base + i * CHUNK, CHUNK)]
            )

    return gather_k


def kernel(edge_types, table, W1, b1, W2, b2):
    fused = _fused_table(table, W1, b1, W2, b2)
    idx = edge_types.reshape(B_TOT).astype(jnp.int32)
    out = _make_gather()(idx, fused)
    return out.reshape(B, N, N, HIDDEN_DIM)


# speedup vs baseline: 1.9525x; 1.9525x over previous
import functools

import jax
import jax.numpy as jnp
from jax import lax
from jax.experimental import pallas as pl
from jax.experimental.pallas import tpu as pltpu
from jax.experimental.pallas import tpu_sc as plsc

EDGE_VOCAB = 64
EDGE_DIM = 128
HIDDEN_DIM = 256
B, N = 16, 64
B_TOT = B * N * N


def _mlp_table_kernel(table_ref, w1_ref, b1_ref, w2_ref, b2_ref, out_ref):
    h = jnp.dot(table_ref[...], w1_ref[...], preferred_element_type=jnp.float32)
    h = h + b1_ref[...]
    h = jax.nn.gelu(h)
    o = jnp.dot(h, w2_ref[...], preferred_element_type=jnp.float32)
    out_ref[...] = o + b2_ref[...]


N_REPLICAS = 32


def _fused_table(table, W1, b1, W2, b2):
    return pl.pallas_call(
        _mlp_table_kernel,
        grid=(N_REPLICAS,),
        in_specs=[
            pl.BlockSpec((EDGE_VOCAB, EDGE_DIM), lambda i: (0, 0)),
            pl.BlockSpec((EDGE_DIM, HIDDEN_DIM), lambda i: (0, 0)),
            pl.BlockSpec((1, HIDDEN_DIM), lambda i: (0, 0)),
            pl.BlockSpec((HIDDEN_DIM, HIDDEN_DIM), lambda i: (0, 0)),
            pl.BlockSpec((1, HIDDEN_DIM), lambda i: (0, 0)),
        ],
        out_specs=pl.BlockSpec((EDGE_VOCAB, HIDDEN_DIM), lambda i: (i, 0)),
        out_shape=jax.ShapeDtypeStruct(
            (N_REPLICAS * EDGE_VOCAB, HIDDEN_DIM), jnp.float32
        ),
    )(table, W1, b1.reshape(1, HIDDEN_DIM), W2, b2.reshape(1, HIDDEN_DIM))


def _make_gather():
    info = plsc.get_sparse_core_info()
    NC, NS = info.num_cores, info.num_subcores
    NW = NC * NS
    b_per_w = B_TOT // NW
    CHUNK = 128
    n_chunks = b_per_w // CHUNK
    mesh = plsc.VectorSubcoreMesh(core_axis_name="c", subcore_axis_name="s")

    @functools.partial(
        pl.kernel,
        mesh=mesh,
        out_type=jax.ShapeDtypeStruct((B_TOT, HIDDEN_DIM), jnp.float32),
        scratch_types=[
            pltpu.VMEM((b_per_w,), jnp.int32),
            pltpu.VMEM((2, CHUNK, HIDDEN_DIM), jnp.float32),
            pltpu.SemaphoreType.DMA,
            pltpu.SemaphoreType.DMA,
        ],
    )
    def gather_k(idx_hbm, table_hbm, out_hbm, idx_v, rows_v, sem0, sem1):
        wid = lax.axis_index("s") * NC + lax.axis_index("c")
        base = wid * b_per_w
        sems = (sem0, sem1)
        pltpu.sync_copy(idx_hbm.at[pl.ds(base, b_per_w)], idx_v)
        off = (wid * EDGE_VOCAB).astype(jnp.int32)
        for j in range(b_per_w // 16):
            sl = pl.ds(j * 16, 16)
            idx_v[sl] = idx_v[sl] + off

        def start_gather(i):
            return pltpu.async_copy(
                table_hbm.at[idx_v.at[pl.ds(i * CHUNK, CHUNK)]],
                rows_v.at[i % 2],
                sems[i % 2],
            )

        handles = {0: start_gather(0)}
        for i in range(n_chunks):
            handles.pop(i).wait()
            if i + 1 < n_chunks:
                handles[i + 1] = start_gather(i + 1)
            pltpu.sync_copy(
                rows_v.at[i % 2], out_hbm.at[pl.ds(base + i * CHUNK, CHUNK)]
            )

    return gather_k


def kernel(edge_types, table, W1, b1, W2, b2):
    fused = _fused_table(table, W1, b1, W2, b2)
    idx = edge_types.reshape(B_TOT).astype(jnp.int32)
    out = _make_gather()(idx, fused)
    return out.reshape(B, N, N, HIDDEN_DIM)
